# baseline (speedup 1.0000x reference)
import functools

import jax
import jax.numpy as jnp
from jax import lax
from jax.experimental import pallas as pl
from jax.experimental.pallas import tpu as pltpu
from jax.experimental.pallas import tpu_sc as plsc

N_NODES = 100000
DIM = 64

NC = 2
NS = 16
NW = NC * NS

CHUNK = 128
N_CHUNKS = 25
PER_W = CHUNK * N_CHUNKS
N_PAD = PER_W * NW

_mesh = plsc.VectorSubcoreMesh(core_axis_name="c", subcore_axis_name="s")


@functools.partial(
    pl.kernel,
    mesh=_mesh,
    compiler_params=pltpu.CompilerParams(use_tc_tiling_on_sc=False),
    out_type=jax.ShapeDtypeStruct((N_PAD, DIM), jnp.float32),
    scratch_types=[
        pltpu.VMEM((N_CHUNKS, CHUNK), jnp.int32),
        pltpu.VMEM((2, CHUNK, DIM), jnp.float32),
        pltpu.SemaphoreType.DMA,
    ],
)
def _gather_kernel(idx_hbm, table_hbm, out_hbm, idx_v, rows_v, gsem):
    wid = lax.axis_index("s") * NC + lax.axis_index("c")
    base = wid * PER_W
    pltpu.sync_copy(idx_hbm.at[wid], idx_v)

    def body(c, _):
        pltpu.async_copy(table_hbm.at[idx_v.at[c]], rows_v.at[0], gsem).wait()
        pltpu.sync_copy(rows_v.at[0], out_hbm.at[pl.ds(base + c * CHUNK, CHUNK)])
        return 0

    lax.fori_loop(0, N_CHUNKS, body, 0)


def kernel(x, spatial_indexs, table):
    idx = spatial_indexs.astype(jnp.int32)
    idx = jnp.concatenate([idx, jnp.zeros((N_PAD - N_NODES,), jnp.int32)])
    idx3 = idx.reshape(NW, N_CHUNKS, CHUNK)
    out = _gather_kernel(idx3, table)
    return out[:N_NODES][None, None]


# speedup vs baseline: 1.7009x; 1.7009x over previous
import functools

import jax
import jax.numpy as jnp
from jax import lax
from jax.experimental import pallas as pl
from jax.experimental.pallas import tpu as pltpu
from jax.experimental.pallas import tpu_sc as plsc

N_NODES = 100000
DIM = 64

NC = 2
NS = 16
NW = NC * NS

CHUNK = 125
N_CHUNKS = 25
PER_W = CHUNK * N_CHUNKS
NBUF = 4

_mesh = plsc.VectorSubcoreMesh(core_axis_name="c", subcore_axis_name="s")


@functools.partial(
    pl.kernel,
    mesh=_mesh,
    compiler_params=pltpu.CompilerParams(use_tc_tiling_on_sc=False),
    out_type=jax.ShapeDtypeStruct((N_NODES, DIM), jnp.float32),
    scratch_types=[
        pltpu.VMEM((N_CHUNKS, CHUNK), jnp.int32),
        pltpu.VMEM((NBUF, CHUNK, DIM), jnp.float32),
        pltpu.SemaphoreType.DMA,
        pltpu.SemaphoreType.DMA,
    ],
)
def _gather_kernel(idx_hbm, table_hbm, out_hbm, idx_v, rows_v, gsem, wsem):
    wid = lax.axis_index("s") * NC + lax.axis_index("c")
    base = wid * PER_W
    pltpu.sync_copy(idx_hbm.at[wid], idx_v)

    for b in range(NBUF):
        pltpu.async_copy(table_hbm.at[idx_v.at[b]], rows_v.at[b], gsem)

    def body(c, _):
        buf = lax.rem(c, NBUF)
        pltpu.make_async_copy(
            table_hbm.at[idx_v.at[c]], rows_v.at[buf], gsem
        ).wait()
        wr = pltpu.async_copy(
            rows_v.at[buf], out_hbm.at[pl.ds(base + c * CHUNK, CHUNK)], wsem
        )

        @pl.when(c + NBUF < N_CHUNKS)
        def _():
            wr.wait()
            pltpu.async_copy(
                table_hbm.at[idx_v.at[c + NBUF]], rows_v.at[buf], gsem
            )

        return 0

    lax.fori_loop(0, N_CHUNKS, body, 0)

    for b in range(NBUF):
        c = N_CHUNKS - NBUF + b
        pltpu.make_async_copy(
            rows_v.at[c % NBUF], out_hbm.at[pl.ds(base + c * CHUNK, CHUNK)], wsem
        ).wait()


def kernel(x, spatial_indexs, table):
    idx3 = spatial_indexs.astype(jnp.int32).reshape(NW, N_CHUNKS, CHUNK)
    out = _gather_kernel(idx3, table)
    return out[None, None]
